# Initial kernel scaffold; baseline (speedup 1.0000x reference)
import jax
import jax.numpy as jnp
from jax.experimental import pallas as pl


def kernel(x, table, W, b):
    raise NotImplementedError("write your pallas kernel here")


# speedup vs baseline: 7.8285x; 7.8285x over previous
import functools

import jax
import jax.numpy as jnp
from jax import lax
from jax.experimental import pallas as pl
from jax.experimental.pallas import tpu as pltpu
from jax.experimental.pallas import tpu_sc as plsc

B, L = 16384, 200
V, D = 1_000_000, 16
NC, NS = 2, 16
NW = NC * NS
ROWS_PER_W = B // NW
CHUNK_ROWS = 128
N_CHUNKS = ROWS_PER_W // CHUNK_ROWS
IDX_PER_CHUNK = CHUNK_ROWS * L
GROUPS_PER_CHUNK = CHUNK_ROWS // 16


def _tc_head_body(r_ref, m_ref, b_ref, out_ref):
    out_ref[...] = (
        jnp.dot(r_ref[...], m_ref[...], preferred_element_type=jnp.float32)
        + b_ref[0, 0]
    )


def _precompute_scores(table, W, b):
    rt = table.reshape(V // 8, 128)
    m = jnp.kron(jnp.eye(8, dtype=jnp.float32), W.T) * (1.0 / L)
    b2 = (b * (1.0 / L)).reshape(1, 1)
    rb = 5000
    out = pl.pallas_call(
        _tc_head_body,
        grid=(V // 8 // rb,),
        in_specs=[
            pl.BlockSpec((rb, 128), lambda i: (i, 0)),
            pl.BlockSpec((128, 8), lambda i: (0, 0)),
            pl.BlockSpec((1, 1), lambda i: (0, 0), memory_space=pltpu.SMEM),
        ],
        out_specs=pl.BlockSpec((rb, 8), lambda i: (i, 0)),
        out_shape=jax.ShapeDtypeStruct((V // 8, 8), jnp.float32),
    )(rt, m, b2)
    return out.reshape(V)


def _sc_body(t_hbm, xt_hbm, out_hbm, idx_v, vals_v, out_v, sem):
    wid = lax.axis_index("s") * NC + lax.axis_index("c")
    for c in range(N_CHUNKS):
        xt_base = wid * (N_CHUNKS * IDX_PER_CHUNK) + c * IDX_PER_CHUNK
        pltpu.sync_copy(xt_hbm.at[pl.ds(xt_base, IDX_PER_CHUNK)], idx_v)
        pltpu.async_copy(t_hbm.at[idx_v], vals_v, sem).wait()
        for g in range(GROUPS_PER_CHUNK):
            def body(l, acc, g=g):
                return acc + vals_v[pl.ds((g * L + l) * 16, 16)]
            acc = lax.fori_loop(0, L, body, jnp.zeros((16,), jnp.float32))
            y = 1.0 / (1.0 + jnp.exp(-acc))
            z = y * 10000.0 + 0.5
            r = z.astype(jnp.int32).astype(jnp.float32) * (1.0 / 10000.0)
            out_v[pl.ds((c * GROUPS_PER_CHUNK + g) * 16, 16)] = r
    pltpu.sync_copy(out_v, out_hbm.at[pl.ds(wid * ROWS_PER_W, ROWS_PER_W)])


@functools.partial(
    pl.kernel,
    out_type=jax.ShapeDtypeStruct((B,), jnp.float32),
    mesh=plsc.VectorSubcoreMesh(
        core_axis_name="c", subcore_axis_name="s", num_cores=NC, num_subcores=NS
    ),
    scratch_types=[
        pltpu.VMEM((IDX_PER_CHUNK,), jnp.int32),
        pltpu.VMEM((IDX_PER_CHUNK,), jnp.float32),
        pltpu.VMEM((ROWS_PER_W,), jnp.float32),
        pltpu.SemaphoreType.DMA,
    ],
)
def _sc_gather_pool(t_hbm, xt_hbm, out_hbm, idx_v, vals_v, out_v, sem):
    _sc_body(t_hbm, xt_hbm, out_hbm, idx_v, vals_v, out_v, sem)


def kernel(x, table, W, b):
    t = _precompute_scores(table, W, b)
    xt = (
        x.astype(jnp.int32)
        .reshape(B // 16, 16, L)
        .transpose(0, 2, 1)
        .reshape(B * L)
    )
    out = _sc_gather_pool(t, xt)
    return out.reshape(B, 1)


# speedup vs baseline: 8.1398x; 1.0398x over previous
import functools

import jax
import jax.numpy as jnp
from jax import lax
from jax.experimental import pallas as pl
from jax.experimental.pallas import tpu as pltpu
from jax.experimental.pallas import tpu_sc as plsc

B, L = 16384, 200
V, D = 1_000_000, 16
LP = 256
NC, NS = 2, 16
NW = NC * NS
ROWS_PER_W = B // NW
CHUNK_ROWS = 128
N_CHUNKS = ROWS_PER_W // CHUNK_ROWS
IDX_PER_CHUNK = CHUNK_ROWS * L


def _tc_head_body(r_ref, m_ref, b_ref, out_ref):
    out_ref[...] = (
        jnp.dot(r_ref[...], m_ref[...], preferred_element_type=jnp.float32)
        + b_ref[0, 0]
    )


def _precompute_scores(table, W, b):
    rt = table.reshape(V // 8, 128)
    m = jnp.kron(jnp.eye(8, dtype=jnp.float32), W.T) * (1.0 / L)
    b2 = (b * (1.0 / L)).reshape(1, 1)
    rb = 5000
    out = pl.pallas_call(
        _tc_head_body,
        grid=(V // 8 // rb,),
        in_specs=[
            pl.BlockSpec((rb, 128), lambda i: (i, 0)),
            pl.BlockSpec((128, 8), lambda i: (0, 0)),
            pl.BlockSpec((1, 1), lambda i: (0, 0), memory_space=pltpu.SMEM),
        ],
        out_specs=pl.BlockSpec((rb, 8), lambda i: (i, 0)),
        out_shape=jax.ShapeDtypeStruct((V // 8, 8), jnp.float32),
    )(rt, m, b2)
    return out.reshape(V)


def _sc_body(t_hbm, xf_hbm, g_hbm, idx_v, vals_v, sem_g, sem_w):
    wid = lax.axis_index("s") * NC + lax.axis_index("c")
    w_base = wid * (N_CHUNKS * IDX_PER_CHUNK)
    w_row0 = wid * ROWS_PER_W

    def stage(c):
        pltpu.sync_copy(
            xf_hbm.at[pl.ds(w_base + c * IDX_PER_CHUNK, IDX_PER_CHUNK)],
            idx_v[c % 2],
        )
        return pltpu.async_copy(t_hbm.at[idx_v[c % 2]], vals_v[c % 2], sem_g[c % 2])

    def fire_writes(c):
        row0 = w_row0 + c * CHUNK_ROWS

        def body(r, carry):
            pltpu.async_copy(
                vals_v[c % 2].at[pl.ds(r * L, L)],
                g_hbm.at[pl.ds((row0 + r) * LP, L)],
                sem_w[c % 2],
            )
            return carry

        lax.fori_loop(0, CHUNK_ROWS, body, 0)

    def drain_writes(c):
        row0 = w_row0 + c * CHUNK_ROWS

        def body(r, carry):
            pltpu.make_async_copy(
                vals_v[c % 2].at[pl.ds(r * L, L)],
                g_hbm.at[pl.ds((row0 + r) * LP, L)],
                sem_w[c % 2],
            ).wait()
            return carry

        lax.fori_loop(0, CHUNK_ROWS, body, 0)

    cp = stage(0)
    for c in range(N_CHUNKS):
        if c + 1 < N_CHUNKS:
            if c >= 1:
                drain_writes(c - 1)
            nxt = stage(c + 1)
        else:
            nxt = None
        cp.wait()
        cp = nxt
        fire_writes(c)
    drain_writes(N_CHUNKS - 2)
    drain_writes(N_CHUNKS - 1)


@functools.partial(
    pl.kernel,
    out_type=jax.ShapeDtypeStruct((B * LP,), jnp.float32),
    mesh=plsc.VectorSubcoreMesh(
        core_axis_name="c", subcore_axis_name="s", num_cores=NC, num_subcores=NS
    ),
    scratch_types=[
        [pltpu.VMEM((IDX_PER_CHUNK,), jnp.int32) for _ in range(2)],
        [pltpu.VMEM((IDX_PER_CHUNK,), jnp.float32) for _ in range(2)],
        [pltpu.SemaphoreType.DMA for _ in range(2)],
        [pltpu.SemaphoreType.DMA for _ in range(2)],
    ],
)
def _sc_gather(t_hbm, xf_hbm, g_hbm, idx_v, vals_v, sem_g, sem_w):
    _sc_body(t_hbm, xf_hbm, g_hbm, idx_v, vals_v, sem_g, sem_w)


def _tc_pool_body(g_ref, out_ref):
    v = g_ref[...]
    lane = lax.broadcasted_iota(jnp.int32, v.shape, 1)
    s = jnp.sum(jnp.where(lane < L, v, 0.0), axis=1)
    y = 1.0 / (1.0 + jnp.exp(-s))
    out_ref[...] = jnp.round(y * 10000.0) / 10000.0


def _pool_scores(g):
    rb = 2048
    return pl.pallas_call(
        _tc_pool_body,
        grid=(B // rb,),
        in_specs=[pl.BlockSpec((rb, LP), lambda i: (i, 0))],
        out_specs=pl.BlockSpec((rb,), lambda i: (i,)),
        out_shape=jax.ShapeDtypeStruct((B,), jnp.float32),
    )(g)


def kernel(x, table, W, b):
    t = _precompute_scores(table, W, b)
    xf = x.astype(jnp.int32).reshape(B * L)
    g = _sc_gather(t, xf).reshape(B, LP)
    out = _pool_scores(g)
    return out.reshape(B, 1)


# speedup vs baseline: 24.3333x; 2.9894x over previous
import functools

import jax
import jax.numpy as jnp
from jax import lax
from jax.experimental import pallas as pl
from jax.experimental.pallas import tpu as pltpu
from jax.experimental.pallas import tpu_sc as plsc

B, L = 16384, 200
V, D = 1_000_000, 16
LP = 256
NC, NS = 2, 16
NW = NC * NS
ROWS_PER_W = B // NW
CHUNK_ROWS = 128
N_CHUNKS = ROWS_PER_W // CHUNK_ROWS
IDX_PER_CHUNK = CHUNK_ROWS * L


_CB = 65536


def _tc_head_body(tt_ref, w_ref, b_ref, out_ref):
    prod = tt_ref[...] * w_ref[...]
    out_ref[...] = jnp.sum(prod, axis=0) + b_ref[0, 0]


def _precompute_scores(table, W, b):
    tt = table.T
    wc = (W.reshape(16, 1) * (1.0 / L)).astype(jnp.float32)
    b2 = (b * (1.0 / L)).reshape(1, 1)
    return pl.pallas_call(
        _tc_head_body,
        grid=(pl.cdiv(V, _CB),),
        in_specs=[
            pl.BlockSpec((16, _CB), lambda i: (0, i)),
            pl.BlockSpec((16, 1), lambda i: (0, 0)),
            pl.BlockSpec((1, 1), lambda i: (0, 0), memory_space=pltpu.SMEM),
        ],
        out_specs=pl.BlockSpec((_CB,), lambda i: (i,)),
        out_shape=jax.ShapeDtypeStruct((V,), jnp.float32),
    )(tt, wc, b2)


def _sc_body(t_hbm, xf_hbm, g_hbm, idx_v, vals_v, sem_g, sem_w):
    wid = lax.axis_index("s") * NC + lax.axis_index("c")
    w_base = wid * (N_CHUNKS * IDX_PER_CHUNK)
    w_row0 = wid * ROWS_PER_W

    def stage(c):
        pltpu.sync_copy(
            xf_hbm.at[pl.ds(w_base + c * IDX_PER_CHUNK, IDX_PER_CHUNK)],
            idx_v[c % 2],
        )
        return pltpu.async_copy(t_hbm.at[idx_v[c % 2]], vals_v[c % 2], sem_g[c % 2])

    def fire_writes(c):
        row0 = w_row0 + c * CHUNK_ROWS

        def body(r, carry):
            pltpu.async_copy(
                vals_v[c % 2].at[pl.ds(r * L, L)],
                g_hbm.at[pl.ds((row0 + r) * LP, L)],
                sem_w[c % 2],
            )
            return carry

        lax.fori_loop(0, CHUNK_ROWS, body, 0)

    def drain_writes(c):
        row0 = w_row0 + c * CHUNK_ROWS

        def body(r, carry):
            pltpu.make_async_copy(
                vals_v[c % 2].at[pl.ds(r * L, L)],
                g_hbm.at[pl.ds((row0 + r) * LP, L)],
                sem_w[c % 2],
            ).wait()
            return carry

        lax.fori_loop(0, CHUNK_ROWS, body, 0)

    cp = stage(0)
    for c in range(N_CHUNKS):
        if c + 1 < N_CHUNKS:
            if c >= 1:
                drain_writes(c - 1)
            nxt = stage(c + 1)
        else:
            nxt = None
        cp.wait()
        cp = nxt
        fire_writes(c)
    drain_writes(N_CHUNKS - 2)
    drain_writes(N_CHUNKS - 1)


@functools.partial(
    pl.kernel,
    out_type=jax.ShapeDtypeStruct((B * LP,), jnp.float32),
    mesh=plsc.VectorSubcoreMesh(
        core_axis_name="c", subcore_axis_name="s", num_cores=NC, num_subcores=NS
    ),
    scratch_types=[
        [pltpu.VMEM((IDX_PER_CHUNK,), jnp.int32) for _ in range(2)],
        [pltpu.VMEM((IDX_PER_CHUNK,), jnp.float32) for _ in range(2)],
        [pltpu.SemaphoreType.DMA for _ in range(2)],
        [pltpu.SemaphoreType.DMA for _ in range(2)],
    ],
)
def _sc_gather(t_hbm, xf_hbm, g_hbm, idx_v, vals_v, sem_g, sem_w):
    _sc_body(t_hbm, xf_hbm, g_hbm, idx_v, vals_v, sem_g, sem_w)


def _tc_pool_body(g_ref, out_ref):
    v = g_ref[...]
    lane = lax.broadcasted_iota(jnp.int32, v.shape, 1)
    s = jnp.sum(jnp.where(lane < L, v, 0.0), axis=1)
    y = 1.0 / (1.0 + jnp.exp(-s))
    out_ref[...] = jnp.round(y * 10000.0) / 10000.0


def _pool_scores(g):
    rb = 2048
    return pl.pallas_call(
        _tc_pool_body,
        grid=(B // rb,),
        in_specs=[pl.BlockSpec((rb, LP), lambda i: (i, 0))],
        out_specs=pl.BlockSpec((rb,), lambda i: (i,)),
        out_shape=jax.ShapeDtypeStruct((B,), jnp.float32),
    )(g)


def kernel(x, table, W, b):
    t = _precompute_scores(table, W, b)
    xf = x.astype(jnp.int32).reshape(B * L)
    g = _sc_gather(t, xf).reshape(B, LP)
    out = _pool_scores(g)
    return out.reshape(B, 1)
